# baseline (speedup 1.0000x reference)
import functools

import jax
import jax.numpy as jnp
from jax import lax
from jax.experimental import pallas as pl
from jax.experimental.pallas import tpu as pltpu
from jax.experimental.pallas import tpu_sc as plsc

_N = 64
_L = 16
_NCHUNK = _N // _L
_NSTAGE = 8


def _splat(v):
    return jnp.full((_L,), v, jnp.int32)


def _elem(ref, r, c):
    return plsc.load_gather(ref, [_splat(r), _splat(c)])


_W = 8


def _lu_inplace(A, publish, tz):
    lanes = lax.iota(jnp.int32, _L)

    for q in range(_N // _W):
        q0 = q * _W
        cp = q0 // _L
        later = list(range(cp + 1, _NCHUNK))
        urows = {}

        for j in range(_W):
            k = q0 + j
            kd = k + tz
            piv = _elem(A, kd, kd)
            for c in range(cp, _NCHUNK):
                rows = lanes + (c * _L)
                m = rows > k if c == cp else None
                colv = plsc.load_gather(A, [rows, kk := _splat(kd)], mask=m)
                plsc.store_scatter(A, [rows, kk], colv / piv, mask=m)
            urow_p = A[kd, pl.ds(cp * _L, _L)]
            urow_p = jnp.where((lanes + cp * _L) > k, urow_p, 0.0)
            urows[j] = {c: A[kd, pl.ds(c * _L, _L)] for c in later}

            def istep(t, carry2, kd=kd, urow_p=urow_p):
                ia = kd + 1 + 2 * t
                ib_raw = ia + 1
                valid_b = ib_raw < _N
                ib = jnp.where(valid_b, ib_raw, _N - 1)
                la = plsc.load_gather(A, [_splat(ia), _splat(kd)])
                lb = plsc.load_gather(A, [_splat(ib), _splat(kd)])
                lb = jnp.where(valid_b, lb, 0.0)
                v = A[ia, pl.ds(cp * _L, _L)]
                A[ia, pl.ds(cp * _L, _L)] = v - la * urow_p
                v = A[ib, pl.ds(cp * _L, _L)]
                A[ib, pl.ds(cp * _L, _L)] = v - lb * urow_p
                return carry2

            lax.fori_loop(0, (_N - k) // 2, istep, 0)
            for j2 in range(j + 1, _W):
                i = q0 + j2 + tz
                li = _elem(A, i, kd)
                for c in later:
                    v = A[i, pl.ds(c * _L, _L)]
                    A[i, pl.ds(c * _L, _L)] = v - li * urows[j][c]
            if q == 0:
                publish(k)

        if later:

            def sstep(t, carry2, q0=q0, later=later, urows=urows):
                ia = q0 + _W + 2 * t
                ib = ia + 1
                for i in (ia, ib):
                    ls = [_elem(A, i, q0 + j + tz) for j in range(_W)]
                    for c in later:
                        acc = A[i, pl.ds(c * _L, _L)]
                        for j in range(_W):
                            acc = acc - ls[j] * urows[j][c]
                        A[i, pl.ds(c * _L, _L)] = acc
                return carry2

            lax.fori_loop(0, (_N - q0 - _W) // 2, sstep, 0)


def _sub_row0(A, row, corr):
    v = A[row, pl.ds(0, _L)]
    A[row, pl.ds(0, _L)] = v - corr


def _sc_lu_pipeline(x):
    mesh = plsc.VectorSubcoreMesh(core_axis_name="c", subcore_axis_name="s")

    @functools.partial(
        pl.kernel,
        out_type=jax.ShapeDtypeStruct((9, _N, _N), jnp.float32),
        mesh=mesh,
        scratch_types=[
            pltpu.VMEM((_N, _N), jnp.float32),
            pltpu.VMEM((_NSTAGE, _N), jnp.float32),
            pltpu.VMEM((_NSTAGE, _N), jnp.float32),
            pltpu.VMEM_SHARED((9, _NSTAGE, _N), jnp.float32),
            pltpu.SMEM((1,), jnp.int32),
            pltpu.SMEM((1,), jnp.int32),
        ],
        compiler_params=pltpu.CompilerParams(needs_layout_passes=False),
    )
    def run(x_hbm, out_hbm, A, S0, S1, stage, flag, zed):
        c = lax.axis_index("c")
        s = lax.axis_index("s")
        lanes = lax.iota(jnp.int32, _L)
        on0 = c == 0
        has_block = on0 & (s <= 8)
        does_lu = has_block & (s != 4)
        lvl1 = on0 & ((s == 3) | (s == 7))
        lvl2 = on0 & (s == 8)

        is_pub = on0 & (s <= 7) & (s != 4)
        pub_k = jnp.where(s == 2, 2, jnp.where(s == 6, 3, 0))
        dst = jnp.where((s == 0) | (s == 3) | (s == 7), 8,
                        jnp.where(s <= 2, 3, 7))

        flag[0] = 0
        zed[0] = 0
        plsc.subcore_barrier()
        tz = zed[0]

        @pl.when(has_block)
        def _():
            pltpu.sync_copy(x_hbm.at[s], A)

        @pl.when(on0 & (s == 4))
        def _():
            pltpu.sync_copy(A, out_hbm.at[4])

        def wait_flag(expected):
            def body(cnt):
                return plsc.fetch_and_add(flag.at[0], 0, subcore_id=s)
            lax.while_loop(lambda cnt: cnt < expected, body, 0)

        @pl.when(lvl1)
        def _():
            wait_flag(2)
            pltpu.sync_copy(stage.at[s - 2], S0)
            pltpu.sync_copy(stage.at[s - 1], S1)
            r = jnp.where(s == 3, 2, 3)
            cols = r + jnp.where(lanes == 1, 1, 0)
            g0 = plsc.load_gather(S1, [_splat(r), cols])
            g1 = plsc.load_gather(S1, [_splat(r + 1), cols])
            p11 = _elem(S0, 1 + tz, 1 + tz)
            corr0 = jnp.where(lanes < 2, g0, 0.0) + jnp.where(lanes == 0, p11, 0.0)
            corr1 = jnp.where(lanes < 2, g1, 0.0)
            _sub_row0(A, 0, corr0)
            _sub_row0(A, 1, corr1)

        @pl.when(lvl2)
        def _():
            wait_flag(3)
            pltpu.sync_copy(stage.at[0], S0)
            pltpu.sync_copy(stage.at[3], S1)
            g = _elem(S0, 1 + tz, 1 + tz) + _elem(S1, 1 + tz, 1 + tz)
            pltpu.sync_copy(stage.at[7], S0)
            g = g + _elem(S0, 1 + tz, 1 + tz)
            _sub_row0(A, 0, jnp.where(lanes == 0, g, 0.0))

        def publish(k):
            @pl.when(is_pub & (k == pub_k))
            def _():
                pltpu.sync_copy(A.at[pl.ds(0, _NSTAGE)], stage.at[s])
                plsc.fetch_and_add(flag.at[0], 1, subcore_id=dst)

        @pl.when(does_lu)
        def _():
            _lu_inplace(A, publish, tz)
            pltpu.sync_copy(A, out_hbm.at[s])

    return run(x)


@jax.jit
def kernel(input):
    return _sc_lu_pipeline(input)


# speedup vs baseline: 2.0529x; 2.0529x over previous
import functools

import jax
import jax.numpy as jnp
from jax import lax
from jax.experimental import pallas as pl
from jax.experimental.pallas import tpu as pltpu
from jax.experimental.pallas import tpu_sc as plsc

_N = 64
_L = 16
_NCHUNK = _N // _L
_NSTAGE = 8


def _splat(v):
    return jnp.full((_L,), v, jnp.int32)


def _elem(ref, r, c):
    return plsc.load_gather(ref, [_splat(r), _splat(c)])


def _lu_inplace(A, publish):
    lanes = lax.iota(jnp.int32, _L)

    def make_kstep(p):
        chunks = list(range(p, _NCHUNK))

        def kstep(k, carry):
            kk = _splat(k)
            piv = plsc.load_gather(A, [kk, kk])
            for c in chunks:
                rows = lanes + (c * _L)
                m = rows > k if c == p else None
                colv = plsc.load_gather(A, [rows, kk], mask=m)
                plsc.store_scatter(A, [rows, kk], colv / piv, mask=m)
            urow = {}
            for c in chunks:
                rv = A[k, pl.ds(c * _L, _L)]
                if c == p:
                    rv = jnp.where((lanes + (c * _L)) > k, rv, 0.0)
                urow[c] = rv

            @pl.when((k % 2) == 0)
            def _():
                i = k + 1
                l = plsc.load_gather(A, [_splat(i), kk])
                for c in chunks:
                    v = A[i, pl.ds(c * _L, _L)]
                    A[i, pl.ds(c * _L, _L)] = v - l * urow[c]

            base = k + 1 + ((k + 1) & 1)
            npairs = (_N - 1 - k) // 2

            @functools.partial(plsc.parallel_loop, 0, npairs, unroll=2)
            def _(t):
                ia = base + 2 * t
                ib = ia + 1
                la = plsc.load_gather(A, [_splat(ia), kk])
                lb = plsc.load_gather(A, [_splat(ib), kk])
                for c in chunks:
                    v = A[ia, pl.ds(c * _L, _L)]
                    A[ia, pl.ds(c * _L, _L)] = v - la * urow[c]
                for c in chunks:
                    v = A[ib, pl.ds(c * _L, _L)]
                    A[ib, pl.ds(c * _L, _L)] = v - lb * urow[c]
            if p == 0:
                publish(k)
            return carry

        return kstep

    for p in range(_NCHUNK):
        lax.fori_loop(p * _L, (p + 1) * _L, make_kstep(p), 0)


def _sub_row0(A, row, corr):
    v = A[row, pl.ds(0, _L)]
    A[row, pl.ds(0, _L)] = v - corr


def _sc_lu_pipeline(x):
    mesh = plsc.VectorSubcoreMesh(core_axis_name="c", subcore_axis_name="s")

    @functools.partial(
        pl.kernel,
        out_type=jax.ShapeDtypeStruct((9, _N, _N), jnp.float32),
        mesh=mesh,
        scratch_types=[
            pltpu.VMEM((_N, _N), jnp.float32),
            pltpu.VMEM((_NSTAGE, _N), jnp.float32),
            pltpu.VMEM((_NSTAGE, _N), jnp.float32),
            pltpu.VMEM_SHARED((9, _NSTAGE, _N), jnp.float32),
            pltpu.SMEM((1,), jnp.int32),
        ],
        compiler_params=pltpu.CompilerParams(needs_layout_passes=False),
    )
    def run(x_hbm, out_hbm, A, S0, S1, stage, flag):
        c = lax.axis_index("c")
        s = lax.axis_index("s")
        lanes = lax.iota(jnp.int32, _L)
        on0 = c == 0
        has_block = on0 & (s <= 8)
        does_lu = has_block & (s != 4)
        lvl1 = on0 & ((s == 3) | (s == 7))
        lvl2 = on0 & (s == 8)

        is_pub = on0 & (s <= 7) & (s != 4)
        pub_k = jnp.where(s == 2, 2, jnp.where(s == 6, 3, 0))
        dst = jnp.where((s == 0) | (s == 3) | (s == 7), 8,
                        jnp.where(s <= 2, 3, 7))

        flag[0] = 0
        plsc.subcore_barrier()

        @pl.when(has_block)
        def _():
            pltpu.sync_copy(x_hbm.at[s], A)

        @pl.when(on0 & (s == 4))
        def _():
            pltpu.sync_copy(A, out_hbm.at[4])

        def wait_flag(expected):
            def body(cnt):
                return plsc.fetch_and_add(flag.at[0], 0, subcore_id=s)
            lax.while_loop(lambda cnt: cnt < expected, body, 0)

        @pl.when(lvl1)
        def _():
            wait_flag(2)
            pltpu.sync_copy(stage.at[s - 2], S0)
            pltpu.sync_copy(stage.at[s - 1], S1)
            r = jnp.where(s == 3, 2, 3)
            cols = r + jnp.where(lanes == 1, 1, 0)
            g0 = plsc.load_gather(S1, [_splat(r), cols])
            g1 = plsc.load_gather(S1, [_splat(r + 1), cols])
            p11 = _elem(S0, 1, 1)
            corr0 = jnp.where(lanes < 2, g0, 0.0) + jnp.where(lanes == 0, p11, 0.0)
            corr1 = jnp.where(lanes < 2, g1, 0.0)
            _sub_row0(A, 0, corr0)
            _sub_row0(A, 1, corr1)

        @pl.when(lvl2)
        def _():
            wait_flag(3)
            pltpu.sync_copy(stage.at[0], S0)
            pltpu.sync_copy(stage.at[3], S1)
            g = _elem(S0, 1, 1) + _elem(S1, 1, 1)
            pltpu.sync_copy(stage.at[7], S0)
            g = g + _elem(S0, 1, 1)
            _sub_row0(A, 0, jnp.where(lanes == 0, g, 0.0))

        def publish(k):
            @pl.when(is_pub & (k == pub_k))
            def _():
                pltpu.sync_copy(A.at[pl.ds(0, _NSTAGE)], stage.at[s])
                plsc.fetch_and_add(flag.at[0], 1, subcore_id=dst)

        @pl.when(does_lu)
        def _():
            _lu_inplace(A, publish)
            pltpu.sync_copy(A, out_hbm.at[s])

    return run(x)


@jax.jit
def kernel(input):
    return _sc_lu_pipeline(input)
